# baseline (speedup 1.0000x reference)
import functools

import jax
import jax.numpy as jnp
from jax import lax
from jax.experimental import pallas as pl
from jax.experimental.pallas import tpu as pltpu
from jax.experimental.pallas import tpu_sc as plsc

_B = 16384
_F = 26
_NC = 2
_NS = 16
_NW = _NC * _NS
_RPW = _B // _NW
_IPW = _RPW * _F
_L = 16
_TPAD = 1000448


def _sc_body(table_hbm, idx_hbm, bias_hbm, out_hbm, idx_v, vals_v,
             acc_v, bias_v, sem):
  wid = lax.axis_index("s") * _NC + lax.axis_index("c")
  base = wid * _RPW
  idx_copies = [
      pltpu.async_copy(
          idx_hbm.at[f, pl.ds(base, _RPW)],
          idx_v.at[pl.ds(f * _RPW, _RPW)],
          sem,
      )
      for f in range(_F)
  ]
  pltpu.sync_copy(bias_hbm, bias_v)
  for c in idx_copies:
    c.wait()
  pltpu.async_copy(table_hbm.at[idx_v], vals_v, sem).wait()
  bias_reg = bias_v[...]

  def chunk_body(c, _):
    off = c * _L
    acc = bias_reg + vals_v[pl.ds(off, _L)]
    for f in range(1, _F):
      acc = acc + vals_v[pl.ds(f * _RPW + off, _L)]
    acc_v[pl.ds(off, _L)] = acc
    return 0

  lax.fori_loop(0, _RPW // _L, chunk_body, 0)
  pltpu.sync_copy(acc_v, out_hbm.at[pl.ds(base, _RPW)])


@functools.partial(
    pl.kernel,
    out_type=jax.ShapeDtypeStruct((_B,), jnp.float32),
    mesh=plsc.VectorSubcoreMesh(core_axis_name="c", subcore_axis_name="s"),
    scratch_types=[
        pltpu.VMEM((_IPW,), jnp.int32),
        pltpu.VMEM((_IPW,), jnp.float32),
        pltpu.VMEM((_RPW,), jnp.float32),
        pltpu.VMEM((_L,), jnp.float32),
        pltpu.SemaphoreType.DMA,
    ],
)
def _sc_kernel(table, idxp, bias16, out, idx_v, vals_v, acc_v, bias_v, sem):
  _sc_body(table, idxp, bias16, out, idx_v, vals_v, acc_v, bias_v, sem)


@jax.jit
def kernel(x, fc_weight, bias):
  table = jnp.pad(fc_weight, ((0, _TPAD - 1000000), (0, 0))).reshape(-1)
  idxp = x.astype(jnp.int32).T
  bias16 = jnp.broadcast_to(bias.reshape(-1)[:1], (_L,)).astype(jnp.float32)
  out = _sc_kernel(table, idxp, bias16)
  return out.reshape(_B, 1)


# speedup vs baseline: 1.0038x; 1.0038x over previous
import functools

import jax
import jax.numpy as jnp
from jax import lax
from jax.experimental import pallas as pl
from jax.experimental.pallas import tpu as pltpu
from jax.experimental.pallas import tpu_sc as plsc

_B = 16384
_F = 26
_NC = 2
_NS = 16
_NW = _NC * _NS
_RPW = _B // _NW
_IPW = _RPW * _F
_L = 16
_TPAD = 1000448


def _sc_body(table_hbm, idx_hbm, bias_hbm, out_hbm, idx_v, vals_v,
             acc_v, bias_v, sem):
  wid = lax.axis_index("s") * _NC + lax.axis_index("c")
  base = wid * _RPW
  idx_copies = [
      pltpu.async_copy(
          idx_hbm.at[f, pl.ds(base, _RPW)],
          idx_v.at[pl.ds(f * _RPW, _RPW)],
          sem,
      )
      for f in range(_F)
  ]
  pltpu.sync_copy(bias_hbm, bias_v)
  for c in idx_copies:
    c.wait()
  pltpu.async_copy(table_hbm.at[idx_v], vals_v, sem).wait()
  bias_reg = bias_v[...]

  def chunk_body(c, _):
    off = c * _L
    acc = bias_reg + vals_v[pl.ds(off, _L)]
    for f in range(1, _F):
      acc = acc + vals_v[pl.ds(f * _RPW + off, _L)]
    acc_v[pl.ds(off, _L)] = acc
    return 0

  lax.fori_loop(0, _RPW // _L, chunk_body, 0)
  pltpu.sync_copy(acc_v, out_hbm.at[pl.ds(base, _RPW)])


@functools.partial(
    pl.kernel,
    out_type=jax.ShapeDtypeStruct((_B,), jnp.float32),
    mesh=plsc.VectorSubcoreMesh(core_axis_name="c", subcore_axis_name="s"),
    scratch_types=[
        pltpu.VMEM((_IPW,), jnp.int32),
        pltpu.VMEM((_IPW,), jnp.float32),
        pltpu.VMEM((_RPW,), jnp.float32),
        pltpu.VMEM((_L,), jnp.float32),
        pltpu.SemaphoreType.DMA,
    ],
)
def _sc_kernel(table, idxp, bias16, out, idx_v, vals_v, acc_v, bias_v, sem):
  _sc_body(table, idxp, bias16, out, idx_v, vals_v, acc_v, bias_v, sem)


@jax.jit
def kernel(x, fc_weight, bias):
  table = jnp.pad(fc_weight, ((0, _TPAD - fc_weight.shape[0]), (0, 0))).reshape(-1)
  idxp = x.astype(jnp.int32).T
  bias16 = jnp.broadcast_to(bias.reshape(-1)[:1], (_L,)).astype(jnp.float32)
  out = _sc_kernel(table, idxp, bias16)
  return out.reshape(_B, 1)
